# baseline (speedup 1.0000x reference)
import functools

import jax
import jax.numpy as jnp
import numpy as np
from jax import lax
from jax.experimental import pallas as pl
from jax.experimental.pallas import tpu as pltpu
from jax.experimental.pallas import tpu_sc as plsc

N = 10000
E = 320000
D = 128
DO = 16

NC = 2
NS = 16
NW = NC * NS

NPADD = 10240
RPTD = NPADD // NS
NPAD = 10112
RPT = NPAD // NS
CHUNK = 128
NCHUNK = 81
EPT = NCHUNK * CHUNK
EPAD = EPT * NW
NBUF = 3

_sc_mesh = plsc.VectorSubcoreMesh(core_axis_name="c", subcore_axis_name="s")

_PAD = EPAD - E
_ROWPAD = np.arange(_PAD, dtype=np.int32) % N
_COLPAD = N + (np.arange(_PAD, dtype=np.int32) % (NPAD - N))
_ONES_C = np.ones((CHUNK,), np.float32)
_Z1 = np.zeros((NPADD,), np.float32)
_Z2 = np.zeros((NPAD, D), np.float32)


EPT_D = E // NW
NCHUNK_D = EPT_D // CHUNK
TAIL_D = EPT_D - NCHUNK_D * CHUNK


@functools.partial(
    pl.kernel,
    out_type=jax.ShapeDtypeStruct((NC * NPADD,), jnp.float32),
    mesh=_sc_mesh,
    scratch_types=[
        pltpu.VMEM((2, CHUNK), jnp.int32),
        pltpu.VMEM((1, TAIL_D), jnp.int32),
        pltpu.VMEM((CHUNK,), jnp.float32),
        pltpu.VMEM_SHARED((NPADD,), jnp.float32),
        pltpu.SemaphoreType.DMA,
        pltpu.SemaphoreType.DMA,
    ],
)
def _deg_kernel(col_hbm, ones_hbm, z1_hbm, out_hbm, cidx_d, tidx_d, ones_v,
                acc, dsem0, dsem1):
    dsems = (dsem0, dsem1)
    c = lax.axis_index("c")
    s = lax.axis_index("s")
    base = (c * NS + s) * EPT_D
    pltpu.sync_copy(ones_hbm, ones_v)
    pltpu.sync_copy(z1_hbm.at[pl.ds(s * RPTD, RPTD)],
                    acc.at[pl.ds(s * RPTD, RPTD)])
    plsc.subcore_barrier()

    def issue_cidx(j, b):
        pltpu.async_copy(col_hbm.at[pl.ds(base + j * CHUNK, CHUNK)],
                         cidx_d.at[b], dsems[b])

    def wait_cidx(b):
        pltpu.make_async_copy(col_hbm.at[pl.ds(0, CHUNK)], cidx_d.at[b],
                              dsems[b]).wait()

    issue_cidx(0, 0)
    issue_cidx(1, 1)

    def substep_d(j, b):
        wait_cidx(b)
        pltpu.sync_copy(ones_v, acc.at[cidx_d.at[b]], add=True)

        @pl.when(j + 2 < NCHUNK_D)
        def _():
            issue_cidx(j + 2, b)

    def step(t, carry):
        substep_d(2 * t, 0)
        substep_d(2 * t + 1, 1)
        return carry

    lax.fori_loop(0, NCHUNK_D // 2, step, 0)
    pltpu.sync_copy(col_hbm.at[pl.ds(base + NCHUNK_D * CHUNK, TAIL_D)],
                    tidx_d.at[0])
    pltpu.sync_copy(ones_v.at[pl.ds(0, TAIL_D)], acc.at[tidx_d.at[0]],
                    add=True)
    plsc.subcore_barrier()
    pltpu.sync_copy(acc.at[pl.ds(s * RPTD, RPTD)],
                    out_hbm.at[pl.ds(c * NPADD + s * RPTD, RPTD)])


@functools.partial(
    pl.kernel,
    out_type=jax.ShapeDtypeStruct((NC * NPAD, D), jnp.float32),
    mesh=_sc_mesh,
    scratch_types=[
        pltpu.VMEM((NBUF, CHUNK), jnp.int32),
        pltpu.VMEM((NBUF, CHUNK), jnp.int32),
        pltpu.VMEM((NBUF, CHUNK, D), jnp.float32),
        pltpu.VMEM_SHARED((NPAD, D), jnp.float32),
        pltpu.SemaphoreType.DMA,
        pltpu.SemaphoreType.DMA,
        pltpu.SemaphoreType.DMA,
        pltpu.SemaphoreType.DMA,
        pltpu.SemaphoreType.DMA,
        pltpu.SemaphoreType.DMA,
    ],
)
def _agg_kernel(g_hbm, row_hbm, col_hbm, z2_hbm, out_hbm,
                ridx, cidx, gbufs, acc,
                gsem0, gsem1, gsem2, isem0, isem1, isem2):
    gsems = (gsem0, gsem1, gsem2)
    isems = (isem0, isem1, isem2)
    c = lax.axis_index("c")
    s = lax.axis_index("s")
    base = (c * NS + s) * EPT
    pltpu.sync_copy(z2_hbm.at[pl.ds(s * RPT, RPT)], acc.at[pl.ds(s * RPT, RPT)])
    plsc.subcore_barrier()

    def issue_idx(j, b):
        pltpu.async_copy(row_hbm.at[pl.ds(base + j * CHUNK, CHUNK)],
                         ridx.at[b], isems[b])
        pltpu.async_copy(col_hbm.at[pl.ds(base + j * CHUNK, CHUNK)],
                         cidx.at[b], isems[b])

    def wait_idx(b):
        pltpu.make_async_copy(row_hbm.at[pl.ds(0, CHUNK)], ridx.at[b],
                              isems[b]).wait()
        pltpu.make_async_copy(col_hbm.at[pl.ds(0, CHUNK)], cidx.at[b],
                              isems[b]).wait()

    def issue_gather(b):
        pltpu.async_copy(g_hbm.at[ridx.at[b]], gbufs.at[b], gsems[b])

    def wait_gather(b):
        pltpu.make_async_copy(g_hbm.at[ridx.at[0]], gbufs.at[b],
                              gsems[b]).wait()

    for b in range(NBUF):
        issue_idx(b, b)
    wait_idx(0)
    issue_gather(0)

    def substep(j, b):
        bn = (b + 1) % NBUF

        @pl.when(j + 1 < NCHUNK)
        def _():
            wait_idx(bn)
            issue_gather(bn)

        wait_gather(b)
        pltpu.sync_copy(gbufs.at[b], acc.at[cidx.at[b]], add=True)

        @pl.when(j + NBUF < NCHUNK)
        def _():
            issue_idx(j + NBUF, b)

    def outer(t, carry):
        for b in range(NBUF):
            substep(NBUF * t + b, b)
        return carry

    lax.fori_loop(0, NCHUNK // NBUF, outer, 0)
    plsc.subcore_barrier()
    pltpu.sync_copy(acc.at[pl.ds(s * RPT, RPT)],
                    out_hbm.at[pl.ds(c * NPAD + s * RPT, RPT)])


BLK = 1000
_GRID = (N // BLK,)


def _k1m_body(x_ref, w_ref, out_ref):
    out_ref[...] = jnp.dot(x_ref[...], w_ref[...],
                           preferred_element_type=jnp.float32)


_k1m = pl.pallas_call(
    _k1m_body,
    grid=_GRID,
    in_specs=[
        pl.BlockSpec((BLK, D), lambda i: (i, 0)),
        pl.BlockSpec((D, D), lambda i: (0, 0)),
    ],
    out_specs=pl.BlockSpec((BLK, D), lambda i: (i, 0)),
    out_shape=jax.ShapeDtypeStruct((N, D), jnp.float32),
)


def _ksc_body(h_ref, degc_ref, out_ref):
    dis = lax.rsqrt(degc_ref[...])
    out_ref[...] = dis * h_ref[...]


_ksc = pl.pallas_call(
    _ksc_body,
    grid=_GRID,
    in_specs=[
        pl.BlockSpec((BLK, D), lambda i: (i, 0)),
        pl.BlockSpec((BLK, 1), lambda i: (i, 0)),
    ],
    out_specs=pl.BlockSpec((BLK, D), lambda i: (i, 0)),
    out_shape=jax.ShapeDtypeStruct((N, D), jnp.float32),
)


def _k2_body(p_ref, g_ref, degc_ref, b_ref, w_ref, out_ref):
    dis = lax.rsqrt(degc_ref[...])
    p = p_ref[...]
    act = jnp.maximum(dis * (p[0] + p[1] + g_ref[...]) + b_ref[...], 0.0)
    out_ref[...] = dis * jnp.dot(act, w_ref[...],
                                 preferred_element_type=jnp.float32)


_k2 = pl.pallas_call(
    _k2_body,
    grid=_GRID,
    in_specs=[
        pl.BlockSpec((NC, BLK, D), lambda i: (0, i, 0)),
        pl.BlockSpec((BLK, D), lambda i: (i, 0)),
        pl.BlockSpec((BLK, 1), lambda i: (i, 0)),
        pl.BlockSpec((1, D), lambda i: (0, 0)),
        pl.BlockSpec((D, D), lambda i: (0, 0)),
    ],
    out_specs=pl.BlockSpec((BLK, D), lambda i: (i, 0)),
    out_shape=jax.ShapeDtypeStruct((N, D), jnp.float32),
)


def _k4_body(p_ref, g_ref, degc_ref, b_ref, wf1_ref, bf1_ref, wf2_ref,
             bf2_ref, out_ref):
    dis = lax.rsqrt(degc_ref[...])
    p = p_ref[...]
    act = jnp.maximum(dis * (p[0] + p[1] + g_ref[...]) + b_ref[...], 0.0)
    t = jnp.maximum(jnp.dot(act, wf1_ref[...],
                            preferred_element_type=jnp.float32) + bf1_ref[...],
                    0.0)
    out_ref[...] = jnp.dot(t, wf2_ref[...],
                           preferred_element_type=jnp.float32) + bf2_ref[...]


_k4 = pl.pallas_call(
    _k4_body,
    grid=_GRID,
    in_specs=[
        pl.BlockSpec((NC, BLK, D), lambda i: (0, i, 0)),
        pl.BlockSpec((BLK, D), lambda i: (i, 0)),
        pl.BlockSpec((BLK, 1), lambda i: (i, 0)),
        pl.BlockSpec((1, D), lambda i: (0, 0)),
        pl.BlockSpec((D, D), lambda i: (0, 0)),
        pl.BlockSpec((1, D), lambda i: (0, 0)),
        pl.BlockSpec((D, DO), lambda i: (0, 0)),
        pl.BlockSpec((1, DO), lambda i: (0, 0)),
    ],
    out_specs=pl.BlockSpec((BLK, DO), lambda i: (i, 0)),
    out_shape=jax.ShapeDtypeStruct((N, DO), jnp.float32),
)


def kernel(x, edge_index, W1, b1, W2, b2, W3, b3, Wf1, bf1, Wf2, bf2):
    row = edge_index[0]
    col = edge_index[1]
    row_p = jnp.concatenate([row, _ROWPAD])
    col_p = jnp.concatenate([col, _COLPAD])
    ones_c = jnp.asarray(_ONES_C)
    z1 = jnp.asarray(_Z1)
    z2 = jnp.asarray(_Z2)

    degp = _deg_kernel(col, ones_c, z1)
    h1 = _k1m(x, W1)
    degc = (1.0 + degp[:N] + degp[NPADD:NPADD + N])[:, None]

    g1 = _ksc(h1, degc)
    p1 = _agg_kernel(g1, row_p, col_p, z2).reshape(NC, NPAD, D)
    g2 = _k2(p1, g1, degc, b1.reshape(1, D), W2)
    p2 = _agg_kernel(g2, row_p, col_p, z2).reshape(NC, NPAD, D)
    g3 = _k2(p2, g2, degc, b2.reshape(1, D), W3)
    p3 = _agg_kernel(g3, row_p, col_p, z2).reshape(NC, NPAD, D)
    pred = _k4(p3, g3, degc, b3.reshape(1, D), Wf1, bf1.reshape(1, D),
               Wf2, bf2.reshape(1, DO))
    return pred


# speedup vs baseline: 1.0203x; 1.0203x over previous
import functools

import jax
import jax.numpy as jnp
import numpy as np
from jax import lax
from jax.experimental import pallas as pl
from jax.experimental.pallas import tpu as pltpu
from jax.experimental.pallas import tpu_sc as plsc

N = 10000
E = 320000
D = 128
DO = 16

NC = 2
NS = 16
NW = NC * NS

NPADD = 10240
RPTD = NPADD // NS
NPAD = 10112
RPT = NPAD // NS
CHUNK = 128
NBUF = 3

_sc_mesh = plsc.VectorSubcoreMesh(core_axis_name="c", subcore_axis_name="s")

_ONES_C = np.ones((CHUNK,), np.float32)
_Z1 = np.zeros((RPTD,), np.float32)
_Z2 = np.zeros((RPT, D), np.float32)


NCH_TOT = E // CHUNK


@functools.partial(
    pl.kernel,
    out_type=jax.ShapeDtypeStruct((NC * NPADD,), jnp.float32),
    mesh=_sc_mesh,
    scratch_types=[
        pltpu.VMEM((2, CHUNK), jnp.int32),
        pltpu.VMEM((CHUNK,), jnp.float32),
        pltpu.VMEM_SHARED((NPADD,), jnp.float32),
        pltpu.SemaphoreType.DMA,
        pltpu.SemaphoreType.DMA,
    ],
)
def _deg_kernel(col_hbm, ones_hbm, z1_hbm, out_hbm, cidx_d, ones_v,
                acc, dsem0, dsem1):
    dsems = (dsem0, dsem1)
    c = lax.axis_index("c")
    s = lax.axis_index("s")
    wid = c * NS + s
    start = wid * NCH_TOT // NW
    cnt = (wid + 1) * NCH_TOT // NW - start
    base = start * CHUNK
    pltpu.sync_copy(ones_hbm, ones_v)
    pltpu.sync_copy(z1_hbm, acc.at[pl.ds(s * RPTD, RPTD)])
    plsc.subcore_barrier()

    def issue_cidx(j, b):
        pltpu.async_copy(col_hbm.at[pl.ds(base + j * CHUNK, CHUNK)],
                         cidx_d.at[b], dsems[b])

    def wait_cidx(b):
        pltpu.make_async_copy(col_hbm.at[pl.ds(0, CHUNK)], cidx_d.at[b],
                              dsems[b]).wait()

    issue_cidx(0, 0)
    issue_cidx(1, 1)

    def substep_d(j, b):
        @pl.when(j < cnt)
        def _():
            wait_cidx(b)
            pltpu.sync_copy(ones_v, acc.at[cidx_d.at[b]], add=True)

        @pl.when(j + 2 < cnt)
        def _():
            issue_cidx(j + 2, b)

    def step(t, carry):
        substep_d(2 * t, 0)
        substep_d(2 * t + 1, 1)
        return carry

    lax.fori_loop(0, (NCH_TOT // NW + 2) // 2, step, 0)
    plsc.subcore_barrier()
    pltpu.sync_copy(acc.at[pl.ds(s * RPTD, RPTD)],
                    out_hbm.at[pl.ds(c * NPADD + s * RPTD, RPTD)])


@functools.partial(
    pl.kernel,
    out_type=jax.ShapeDtypeStruct((NC * NPAD, D), jnp.float32),
    mesh=_sc_mesh,
    scratch_types=[
        pltpu.VMEM((NBUF, CHUNK), jnp.int32),
        pltpu.VMEM((NBUF, CHUNK), jnp.int32),
        pltpu.VMEM((NBUF, CHUNK, D), jnp.float32),
        pltpu.VMEM_SHARED((NPAD, D), jnp.float32),
        pltpu.SemaphoreType.DMA,
        pltpu.SemaphoreType.DMA,
        pltpu.SemaphoreType.DMA,
        pltpu.SemaphoreType.DMA,
        pltpu.SemaphoreType.DMA,
        pltpu.SemaphoreType.DMA,
    ],
)
def _agg_kernel(g_hbm, row_hbm, col_hbm, z2_hbm, out_hbm,
                ridx, cidx, gbufs, acc,
                gsem0, gsem1, gsem2, isem0, isem1, isem2):
    gsems = (gsem0, gsem1, gsem2)
    isems = (isem0, isem1, isem2)
    c = lax.axis_index("c")
    s = lax.axis_index("s")
    wid = c * NS + s
    start = wid * NCH_TOT // NW
    cnt = (wid + 1) * NCH_TOT // NW - start
    base = start * CHUNK
    pltpu.sync_copy(z2_hbm, acc.at[pl.ds(s * RPT, RPT)])
    plsc.subcore_barrier()

    def issue_idx(j, b):
        pltpu.async_copy(row_hbm.at[pl.ds(base + j * CHUNK, CHUNK)],
                         ridx.at[b], isems[b])
        pltpu.async_copy(col_hbm.at[pl.ds(base + j * CHUNK, CHUNK)],
                         cidx.at[b], isems[b])

    def wait_idx(b):
        pltpu.make_async_copy(row_hbm.at[pl.ds(0, CHUNK)], ridx.at[b],
                              isems[b]).wait()
        pltpu.make_async_copy(col_hbm.at[pl.ds(0, CHUNK)], cidx.at[b],
                              isems[b]).wait()

    def issue_gather(b):
        pltpu.async_copy(g_hbm.at[ridx.at[b]], gbufs.at[b], gsems[b])

    def wait_gather(b):
        pltpu.make_async_copy(g_hbm.at[ridx.at[0]], gbufs.at[b],
                              gsems[b]).wait()

    for b in range(NBUF):
        issue_idx(b, b)
    wait_idx(0)
    issue_gather(0)

    def substep(j, b):
        bn = (b + 1) % NBUF

        @pl.when(j + 1 < cnt)
        def _():
            wait_idx(bn)
            issue_gather(bn)

        @pl.when(j < cnt)
        def _():
            wait_gather(b)
            pltpu.sync_copy(gbufs.at[b], acc.at[cidx.at[b]], add=True)

        @pl.when(j + NBUF < cnt)
        def _():
            issue_idx(j + NBUF, b)

    def outer(t, carry):
        for b in range(NBUF):
            substep(NBUF * t + b, b)
        return carry

    lax.fori_loop(0, (NCH_TOT // NW + NBUF) // NBUF, outer, 0)
    plsc.subcore_barrier()
    pltpu.sync_copy(acc.at[pl.ds(s * RPT, RPT)],
                    out_hbm.at[pl.ds(c * NPAD + s * RPT, RPT)])


BLK = 1000
_GRID = (N // BLK,)


def _k1m_body(x_ref, w_ref, out_ref):
    out_ref[...] = jnp.dot(x_ref[...], w_ref[...],
                           preferred_element_type=jnp.float32)


_k1m = pl.pallas_call(
    _k1m_body,
    grid=_GRID,
    in_specs=[
        pl.BlockSpec((BLK, D), lambda i: (i, 0)),
        pl.BlockSpec((D, D), lambda i: (0, 0)),
    ],
    out_specs=pl.BlockSpec((BLK, D), lambda i: (i, 0)),
    out_shape=jax.ShapeDtypeStruct((N, D), jnp.float32),
)


def _ksc_body(h_ref, degc_ref, out_ref):
    dis = lax.rsqrt(degc_ref[...])
    out_ref[...] = dis * h_ref[...]


_ksc = pl.pallas_call(
    _ksc_body,
    grid=_GRID,
    in_specs=[
        pl.BlockSpec((BLK, D), lambda i: (i, 0)),
        pl.BlockSpec((BLK, 1), lambda i: (i, 0)),
    ],
    out_specs=pl.BlockSpec((BLK, D), lambda i: (i, 0)),
    out_shape=jax.ShapeDtypeStruct((N, D), jnp.float32),
)


def _k2_body(p_ref, g_ref, degc_ref, b_ref, w_ref, out_ref):
    dis = lax.rsqrt(degc_ref[...])
    p = p_ref[...]
    act = jnp.maximum(dis * (p[0] + p[1] + g_ref[...]) + b_ref[...], 0.0)
    out_ref[...] = dis * jnp.dot(act, w_ref[...],
                                 preferred_element_type=jnp.float32)


_k2 = pl.pallas_call(
    _k2_body,
    grid=_GRID,
    in_specs=[
        pl.BlockSpec((NC, BLK, D), lambda i: (0, i, 0)),
        pl.BlockSpec((BLK, D), lambda i: (i, 0)),
        pl.BlockSpec((BLK, 1), lambda i: (i, 0)),
        pl.BlockSpec((1, D), lambda i: (0, 0)),
        pl.BlockSpec((D, D), lambda i: (0, 0)),
    ],
    out_specs=pl.BlockSpec((BLK, D), lambda i: (i, 0)),
    out_shape=jax.ShapeDtypeStruct((N, D), jnp.float32),
)


def _k4_body(p_ref, g_ref, degc_ref, b_ref, wf1_ref, bf1_ref, wf2_ref,
             bf2_ref, out_ref):
    dis = lax.rsqrt(degc_ref[...])
    p = p_ref[...]
    act = jnp.maximum(dis * (p[0] + p[1] + g_ref[...]) + b_ref[...], 0.0)
    t = jnp.maximum(jnp.dot(act, wf1_ref[...],
                            preferred_element_type=jnp.float32) + bf1_ref[...],
                    0.0)
    out_ref[...] = jnp.dot(t, wf2_ref[...],
                           preferred_element_type=jnp.float32) + bf2_ref[...]


_k4 = pl.pallas_call(
    _k4_body,
    grid=_GRID,
    in_specs=[
        pl.BlockSpec((NC, BLK, D), lambda i: (0, i, 0)),
        pl.BlockSpec((BLK, D), lambda i: (i, 0)),
        pl.BlockSpec((BLK, 1), lambda i: (i, 0)),
        pl.BlockSpec((1, D), lambda i: (0, 0)),
        pl.BlockSpec((D, D), lambda i: (0, 0)),
        pl.BlockSpec((1, D), lambda i: (0, 0)),
        pl.BlockSpec((D, DO), lambda i: (0, 0)),
        pl.BlockSpec((1, DO), lambda i: (0, 0)),
    ],
    out_specs=pl.BlockSpec((BLK, DO), lambda i: (i, 0)),
    out_shape=jax.ShapeDtypeStruct((N, DO), jnp.float32),
)


def kernel(x, edge_index, W1, b1, W2, b2, W3, b3, Wf1, bf1, Wf2, bf2):
    row = edge_index[0]
    col = edge_index[1]
    ones_c = jnp.asarray(_ONES_C)
    z1 = jnp.asarray(_Z1)
    z2 = jnp.asarray(_Z2)

    degp = _deg_kernel(col, ones_c, z1)
    h1 = _k1m(x, W1)
    degc = (1.0 + degp[:N] + degp[NPADD:NPADD + N])[:, None]

    g1 = _ksc(h1, degc)
    p1 = _agg_kernel(g1, row, col, z2).reshape(NC, NPAD, D)
    g2 = _k2(p1, g1, degc, b1.reshape(1, D), W2)
    p2 = _agg_kernel(g2, row, col, z2).reshape(NC, NPAD, D)
    g3 = _k2(p2, g2, degc, b2.reshape(1, D), W3)
    p3 = _agg_kernel(g3, row, col, z2).reshape(NC, NPAD, D)
    pred = _k4(p3, g3, degc, b3.reshape(1, D), Wf1, bf1.reshape(1, D),
               Wf2, bf2.reshape(1, DO))
    return pred
